# Initial kernel scaffold; baseline (speedup 1.0000x reference)
import jax
import jax.numpy as jnp
from jax.experimental import pallas as pl


def kernel(x, edge_index, W1, b1, W2, b2, W3, b3, W4, b4):
    raise NotImplementedError("write your pallas kernel here")


# speedup vs baseline: 8.2494x; 8.2494x over previous
import functools

import jax
import jax.numpy as jnp
from jax import lax
from jax.experimental import pallas as pl
from jax.experimental.pallas import tpu as pltpu
from jax.experimental.pallas import tpu_sc as plsc

CH = 128
KB = 4
NSUB = 16
RB = 1000


def _round_up(a, b):
  return (a + b - 1) // b * b


@functools.lru_cache(maxsize=None)
def _make_prop(n, wd, ept, half, tshare):
  acc_rows = NSUB * tshare
  niter = ept // (KB * CH)
  full_tiles = half // tshare
  rem = half - full_tiles * tshare
  mesh = plsc.VectorSubcoreMesh(core_axis_name="c", subcore_axis_name="s",
                                num_cores=2, num_subcores=NSUB)

  @functools.partial(
      pl.kernel,
      out_type=jax.ShapeDtypeStruct((n, wd), jnp.float32),
      mesh=mesh,
      scratch_types=[
          pltpu.VMEM((KB, CH), jnp.int32),
          pltpu.VMEM((KB, CH), jnp.int32),
          pltpu.VMEM((KB, CH), jnp.int32),
          pltpu.VMEM((KB, CH, wd), jnp.float32),
          pltpu.VMEM_SHARED((acc_rows, wd), jnp.float32),
          pltpu.SemaphoreType.DMA,
      ],
      compiler_params=pltpu.CompilerParams(use_tc_tiling_on_sc=False),
  )
  def prop(src_hbm, dst_hbm, g_hbm, zero_hbm, out_hbm,
           src_v, dst_v, loc_v, rows_v, acc, sem):
    cid = lax.axis_index("c")
    sid = lax.axis_index("s")
    base_node = cid * half
    pltpu.sync_copy(zero_hbm, acc.at[pl.ds(sid * tshare, tshare)])
    plsc.subcore_barrier()

    row0 = sid * (ept // CH)

    def body(i, carry):
      r = row0 + i * KB
      pltpu.sync_copy(src_hbm.at[pl.ds(r, KB)], src_v)
      pltpu.sync_copy(dst_hbm.at[pl.ds(r, KB)], dst_v)
      cps = [pltpu.async_copy(g_hbm.at[src_v.at[j]], rows_v.at[j], sem)
             for j in range(KB)]
      for j in range(KB):
        for q in range(CH // 16):
          d = dst_v[j, pl.ds(q * 16, 16)]
          loc = d - base_node
          ok = (loc >= 0) & (loc < half)
          loc_v[j, pl.ds(q * 16, 16)] = jnp.where(ok, loc, half)
      for cp in cps:
        cp.wait()
      for j in range(KB):
        pltpu.sync_copy(rows_v.at[j], acc.at[loc_v.at[j]], add=True)
      return carry

    lax.fori_loop(0, niter, body, 0)
    plsc.subcore_barrier()

    r0 = sid * tshare

    @pl.when(sid < full_tiles)
    def _():
      pltpu.sync_copy(acc.at[pl.ds(r0, tshare)],
                      out_hbm.at[pl.ds(base_node + r0, tshare)])

    if rem:
      @pl.when(sid == full_tiles)
      def _():
        pltpu.sync_copy(acc.at[pl.ds(r0, rem)],
                        out_hbm.at[pl.ds(base_node + r0, rem)])

  return prop


@functools.lru_cache(maxsize=None)
def _make_deg(n, ept, half, tshare):
  wd = 8
  acc_rows = NSUB * tshare
  niter = ept // (KB * CH)
  full_tiles = half // tshare
  rem = half - full_tiles * tshare
  mesh = plsc.VectorSubcoreMesh(core_axis_name="c", subcore_axis_name="s",
                                num_cores=2, num_subcores=NSUB)

  @functools.partial(
      pl.kernel,
      out_type=jax.ShapeDtypeStruct((n, wd), jnp.float32),
      mesh=mesh,
      scratch_types=[
          pltpu.VMEM((KB, CH), jnp.int32),
          pltpu.VMEM((KB, CH), jnp.int32),
          pltpu.VMEM((CH, wd), jnp.float32),
          pltpu.VMEM_SHARED((acc_rows, wd), jnp.float32),
      ],
      compiler_params=pltpu.CompilerParams(use_tc_tiling_on_sc=False),
  )
  def deg(dst_hbm, ones_hbm, zero_hbm, out_hbm,
          dst_v, loc_v, ones_v, acc):
    cid = lax.axis_index("c")
    sid = lax.axis_index("s")
    base_node = cid * half
    pltpu.sync_copy(ones_hbm, ones_v)
    pltpu.sync_copy(zero_hbm, acc.at[pl.ds(sid * tshare, tshare)])
    plsc.subcore_barrier()

    row0 = sid * (ept // CH)

    def body(i, carry):
      r = row0 + i * KB
      pltpu.sync_copy(dst_hbm.at[pl.ds(r, KB)], dst_v)
      for j in range(KB):
        for q in range(CH // 16):
          d = dst_v[j, pl.ds(q * 16, 16)]
          loc = d - base_node
          ok = (loc >= 0) & (loc < half)
          loc_v[j, pl.ds(q * 16, 16)] = jnp.where(ok, loc, half)
      for j in range(KB):
        pltpu.sync_copy(ones_v, acc.at[loc_v.at[j]], add=True)
      return carry

    lax.fori_loop(0, niter, body, 0)
    plsc.subcore_barrier()

    r0 = sid * tshare

    @pl.when(sid < full_tiles)
    def _():
      pltpu.sync_copy(acc.at[pl.ds(r0, tshare)],
                      out_hbm.at[pl.ds(base_node + r0, tshare)])

    if rem:
      @pl.when(sid == full_tiles)
      def _():
        pltpu.sync_copy(acc.at[pl.ds(r0, rem)],
                        out_hbm.at[pl.ds(base_node + r0, rem)])

  return deg


def _lrelu(z):
  return jnp.where(z >= 0, z, 0.01 * z)


def _row_spec(rb, w):
  return pl.BlockSpec((rb, w), lambda i: (i, 0))


def _full_spec(shape):
  nd = len(shape)
  return pl.BlockSpec(shape, lambda i: (0,) * nd)


def _stage_a(deg8, x8, rb):
  n = deg8.shape[0]

  def body(deg_ref, x_ref, dinv_ref, u1_ref):
    dv = lax.rsqrt(deg_ref[:, 0:1] + 1.0)
    dinv_ref[...] = jnp.broadcast_to(dv, dinv_ref.shape)
    u1_ref[...] = dv * x_ref[...]

  return pl.pallas_call(
      body,
      grid=(n // rb,),
      in_specs=[_row_spec(rb, 8), _row_spec(rb, 8)],
      out_specs=[_row_spec(rb, 8), _row_spec(rb, 8)],
      out_shape=[jax.ShapeDtypeStruct((n, 8), jnp.float32)] * 2,
  )(deg8, x8)


def _stage_b(s1, u1, dinv8, W1p, b1, rb):
  n = s1.shape[0]

  def body(s_ref, u_ref, dv_ref, w_ref, b_ref, oa_ref, ob_ref):
    dv = dv_ref[:, 0:1]
    z = jnp.dot(dv * (s_ref[...] + u_ref[...]), w_ref[...],
                preferred_element_type=jnp.float32) + b_ref[...]
    u2 = dv * _lrelu(z)
    oa_ref[...] = u2[:, :32]
    ob_ref[...] = u2[:, 32:]

  return pl.pallas_call(
      body,
      grid=(n // rb,),
      in_specs=[_row_spec(rb, 8), _row_spec(rb, 8), _row_spec(rb, 8),
                _full_spec((8, 64)), _full_spec((1, 64))],
      out_specs=[_row_spec(rb, 32), _row_spec(rb, 32)],
      out_shape=[jax.ShapeDtypeStruct((n, 32), jnp.float32)] * 2,
  )(s1, u1, dinv8, W1p, b1)


def _stage_c(s2a, s2b, u2a, u2b, dinv8, W2, b2, W3, rb):
  n = s2a.shape[0]

  def body(sa_ref, sb_ref, ua_ref, ub_ref, dv_ref, w2_ref, b2_ref, w3_ref,
           oa_ref, ob_ref):
    dv = dv_ref[:, 0:1]
    va = dv * (sa_ref[...] + ua_ref[...])
    vb = dv * (sb_ref[...] + ub_ref[...])
    v = jnp.concatenate([va, vb], axis=1)
    z2 = jnp.dot(v, w2_ref[...], preferred_element_type=jnp.float32) + b2_ref[...]
    h2 = _lrelu(z2)
    u3 = dv * jnp.dot(h2, w3_ref[...], preferred_element_type=jnp.float32)
    oa_ref[...] = u3[:, :32]
    ob_ref[...] = u3[:, 32:]

  return pl.pallas_call(
      body,
      grid=(n // rb,),
      in_specs=[_row_spec(rb, 32)] * 4 + [_row_spec(rb, 8),
                _full_spec((64, 128)), _full_spec((1, 128)),
                _full_spec((128, 64))],
      out_specs=[_row_spec(rb, 32), _row_spec(rb, 32)],
      out_shape=[jax.ShapeDtypeStruct((n, 32), jnp.float32)] * 2,
  )(s2a, s2b, u2a, u2b, dinv8, W2, b2, W3)


def _stage_d(s3a, s3b, u3a, u3b, dinv8, b3, W4p, rb):
  n = s3a.shape[0]

  def body(sa_ref, sb_ref, ua_ref, ub_ref, dv_ref, b3_ref, w4_ref, o_ref):
    dv = dv_ref[:, 0:1]
    va = dv * (sa_ref[...] + ua_ref[...])
    vb = dv * (sb_ref[...] + ub_ref[...])
    z3 = jnp.concatenate([va, vb], axis=1) + b3_ref[...]
    h3 = _lrelu(z3)
    o_ref[...] = dv * jnp.dot(h3, w4_ref[...],
                              preferred_element_type=jnp.float32)

  return pl.pallas_call(
      body,
      grid=(n // rb,),
      in_specs=[_row_spec(rb, 32)] * 4 + [_row_spec(rb, 8),
                _full_spec((1, 64)), _full_spec((64, 8))],
      out_specs=_row_spec(rb, 8),
      out_shape=jax.ShapeDtypeStruct((n, 8), jnp.float32),
  )(s3a, s3b, u3a, u3b, dinv8, b3, W4p)


def _stage_e(s4p, u4p, dinv8, b4, rb):
  n = s4p.shape[0]

  def body(s_ref, u_ref, dv_ref, b4_ref, o_ref):
    dv = dv_ref[:, 0:1]
    o_ref[...] = dv * (s_ref[:, 0:1] + u_ref[:, 0:1]) + b4_ref[...]

  return pl.pallas_call(
      body,
      grid=(n // rb,),
      in_specs=[_row_spec(rb, 8), _row_spec(rb, 8), _row_spec(rb, 8),
                _full_spec((1, 1))],
      out_specs=_row_spec(rb, 1),
      out_shape=jax.ShapeDtypeStruct((n, 1), jnp.float32),
  )(s4p, u4p, dinv8, b4)


def kernel(x, edge_index, W1, b1, W2, b2, W3, b3, W4, b4):
  n = x.shape[0]
  e = edge_index.shape[1]
  half = n // 2
  tshare = _round_up(-(-(half + 1) // NSUB), 8)
  ept = KB * CH * (-(-e // (NSUB * KB * CH)))
  ep = NSUB * ept

  src0 = edge_index[0]
  dst0 = edge_index[1]
  pad = ep - e
  src_p = jnp.concatenate([src0, jnp.zeros((pad,), jnp.int32)])
  dst_p = jnp.concatenate([dst0, jnp.full((pad,), n, jnp.int32)])
  src2 = src_p.reshape(ep // CH, CH)
  dst2 = dst_p.reshape(ep // CH, CH)

  x8 = jnp.pad(x, ((0, 0), (0, 8 - x.shape[1])))
  W1p = jnp.pad(W1, ((0, 8 - W1.shape[0]), (0, 0)))
  W4p = jnp.pad(W4, ((0, 0), (0, 8 - W4.shape[1])))
  b1r = b1.reshape(1, -1)
  b2r = b2.reshape(1, -1)
  b3r = b3.reshape(1, -1)
  b4r = b4.reshape(1, -1)

  zero8 = jnp.zeros((tshare, 8), jnp.float32)
  zero32 = jnp.zeros((tshare, 32), jnp.float32)
  ones8 = jnp.concatenate(
      [jnp.ones((CH, 1), jnp.float32), jnp.zeros((CH, 7), jnp.float32)], axis=1)

  deg_k = _make_deg(n, ept, half, tshare)
  prop8 = _make_prop(n, 8, ept, half, tshare)
  prop32 = _make_prop(n, 32, ept, half, tshare)

  rb = RB if n % RB == 0 else [d for d in (500, 250, 100, 50, 25, 10, 5, 2, 1)
                               if n % d == 0][0]

  deg8 = deg_k(dst2, ones8, zero8)
  dinv8, u1 = _stage_a(deg8, x8, rb)

  s1 = prop8(src2, dst2, u1, zero8)
  u2a, u2b = _stage_b(s1, u1, dinv8, W1p, b1r, rb)

  s2a = prop32(src2, dst2, u2a, zero32)
  s2b = prop32(src2, dst2, u2b, zero32)
  u3a, u3b = _stage_c(s2a, s2b, u2a, u2b, dinv8, W2, b2r, W3, rb)

  s3a = prop32(src2, dst2, u3a, zero32)
  s3b = prop32(src2, dst2, u3b, zero32)
  u4p = _stage_d(s3a, s3b, u3a, u3b, dinv8, b3r, W4p, rb)

  s4p = prop8(src2, dst2, u4p, zero8)
  return _stage_e(s4p, u4p, dinv8, b4r, rb)
